# Initial kernel scaffold; baseline (speedup 1.0000x reference)
import jax
import jax.numpy as jnp
from jax.experimental import pallas as pl


def kernel(x, edge_index, edge_attr, W1, b1, W2, b2, Ws, bs, U1, c1, gamma, beta, U2, c2):
    raise NotImplementedError("write your pallas kernel here")


# speedup vs baseline: 2.5051x; 2.5051x over previous
import functools

import jax
import jax.numpy as jnp
from jax import lax
from jax.experimental import pallas as pl
from jax.experimental.pallas import tpu as pltpu
from jax.experimental.pallas import tpu_sc as plsc

N = 10000
E = 320000
H = 128

NC = 2
NS = 16
NW = NC * NS
EPW = E // NW
CHUNK = 80
NCHUNK = EPW // CHUNK
RPTA = 632
RPTT = N - (NS - 1) * RPTA

@functools.cache
def _sc_kernels():
    mesh = plsc.VectorSubcoreMesh(core_axis_name="c", subcore_axis_name="s",
                                  num_cores=NC, num_subcores=NS)

    @functools.partial(
        pl.kernel,
        out_type=[
            jax.ShapeDtypeStruct((E, H), jnp.float32),
            jax.ShapeDtypeStruct((E, H), jnp.float32),
        ],
        mesh=mesh,
        scratch_types=[
            pltpu.VMEM((CHUNK,), jnp.int32),
            pltpu.VMEM((CHUNK,), jnp.int32),
            pltpu.VMEM((CHUNK, H), jnp.float32),
            pltpu.VMEM((CHUNK, H), jnp.float32),
            pltpu.SemaphoreType.DMA,
            pltpu.SemaphoreType.DMA,
        ],
    )
    def _sc_gather(p_hbm, q_hbm, src_hbm, dst_hbm, ps_hbm, qd_hbm,
                   idx_s, idx_d, buf_p, buf_q, sem_p, sem_q):
        c = lax.axis_index("c")
        s = lax.axis_index("s")
        base = (s * NC + c) * EPW

        def step(i, carry):
            off = base + i * CHUNK
            pltpu.sync_copy(src_hbm.at[pl.ds(off, CHUNK)], idx_s)
            pltpu.sync_copy(dst_hbm.at[pl.ds(off, CHUNK)], idx_d)
            cp_p = pltpu.async_copy(p_hbm.at[idx_s], buf_p, sem_p)
            cp_q = pltpu.async_copy(q_hbm.at[idx_d], buf_q, sem_q)
            cp_p.wait()
            cp_q.wait()
            pltpu.sync_copy(buf_p, ps_hbm.at[pl.ds(off, CHUNK)])
            pltpu.sync_copy(buf_q, qd_hbm.at[pl.ds(off, CHUNK)])
            return carry

        lax.fori_loop(0, NCHUNK, step, 0)

    @functools.partial(
        pl.kernel,
        out_type=jax.ShapeDtypeStruct((NC, N, H), jnp.float32),
        mesh=mesh,
        scratch_types=[
            pltpu.VMEM((CHUNK,), jnp.int32),
            pltpu.VMEM((CHUNK, H), jnp.float32),
            pltpu.VMEM((8, H), jnp.float32),
            pltpu.VMEM_SHARED((N, H), jnp.float32),
        ],
    )
    def _sc_scatter(mw_hbm, dst_hbm, z_hbm, out_hbm, idx_d, buf, zbuf, acc):
        c = lax.axis_index("c")
        s = lax.axis_index("s")
        base = (s * NC + c) * EPW
        last = s == NS - 1

        pltpu.sync_copy(z_hbm, zbuf)
        nz = jnp.where(last, RPTT // 8, RPTA // 8)

        def zstep(k, carry):
            pltpu.sync_copy(zbuf, acc.at[pl.ds(s * RPTA + k * 8, 8)])
            return carry

        lax.fori_loop(0, nz, zstep, 0)
        plsc.subcore_barrier()

        def step(i, carry):
            off = base + i * CHUNK
            pltpu.sync_copy(dst_hbm.at[pl.ds(off, CHUNK)], idx_d)
            pltpu.sync_copy(mw_hbm.at[pl.ds(off, CHUNK)], buf)
            pltpu.sync_copy(buf, acc.at[idx_d], add=True)
            return carry

        lax.fori_loop(0, NCHUNK, step, 0)
        plsc.subcore_barrier()

        @pl.when(jnp.logical_not(last))
        def _():
            pltpu.sync_copy(acc.at[pl.ds(s * RPTA, RPTA)],
                            out_hbm.at[c, pl.ds(s * RPTA, RPTA)])

        @pl.when(last)
        def _():
            pltpu.sync_copy(acc.at[pl.ds(s * RPTA, RPTT)],
                            out_hbm.at[c, pl.ds(s * RPTA, RPTT)])

    return _sc_gather, _sc_scatter


def _pq_body(x_ref, w1a_ref, w1b_ref, b1_ref, p_ref, q_ref):
    xv = x_ref[...]
    p_ref[...] = (jnp.dot(xv, w1a_ref[...], preferred_element_type=jnp.float32)
                  + b1_ref[...])
    q_ref[...] = jnp.dot(xv, w1b_ref[...], preferred_element_type=jnp.float32)


_pq_call = pl.pallas_call(
    _pq_body,
    out_shape=[
        jax.ShapeDtypeStruct((N, H), jnp.float32),
        jax.ShapeDtypeStruct((N, H), jnp.float32),
    ],
)


BE = 3200


def _edge_body(ps_ref, qd_ref, ea_ref, w1c_ref, w2_ref, b2_ref, wsr_ref,
               bs_ref, dnew_ref, mw_ref):
    ea = ea_ref[...]
    h1 = jnp.maximum(
        ps_ref[...] + qd_ref[...]
        + jnp.dot(ea, w1c_ref[...], preferred_element_type=jnp.float32), 0.0)
    m = jnp.dot(h1, w2_ref[...], preferred_element_type=jnp.float32) + b2_ref[...]
    msg = m * jax.nn.sigmoid(m)
    dnew_ref[...] = ea + msg
    ew = jax.nn.sigmoid(jnp.sum(msg * wsr_ref[...], axis=1, keepdims=True)
                        + bs_ref[...])
    mw_ref[...] = msg * ew


_edge_call = pl.pallas_call(
    _edge_body,
    grid=(E // BE,),
    in_specs=[
        pl.BlockSpec((BE, H), lambda i: (i, 0)),
        pl.BlockSpec((BE, H), lambda i: (i, 0)),
        pl.BlockSpec((BE, H), lambda i: (i, 0)),
        pl.BlockSpec((H, H), lambda i: (0, 0)),
        pl.BlockSpec((H, H), lambda i: (0, 0)),
        pl.BlockSpec((1, H), lambda i: (0, 0)),
        pl.BlockSpec((1, H), lambda i: (0, 0)),
        pl.BlockSpec((1, 1), lambda i: (0, 0)),
    ],
    out_specs=[
        pl.BlockSpec((BE, H), lambda i: (i, 0)),
        pl.BlockSpec((BE, H), lambda i: (i, 0)),
    ],
    out_shape=[
        jax.ShapeDtypeStruct((E, H), jnp.float32),
        jax.ShapeDtypeStruct((E, H), jnp.float32),
    ],
)


def _update_body(part_ref, x_ref, u1_ref, c1_ref, g_ref, b_ref, u2_ref,
                 c2_ref, out_ref):
    xv = x_ref[...]
    u_in = part_ref[0] + part_ref[1] + xv
    u1 = jnp.maximum(
        jnp.dot(u_in, u1_ref[...], preferred_element_type=jnp.float32)
        + c1_ref[...], 0.0)
    mean = jnp.mean(u1, axis=0, keepdims=True)
    var = jnp.mean((u1 - mean) ** 2, axis=0, keepdims=True)
    u1n = (u1 - mean) / jnp.sqrt(var + 1e-5) * g_ref[...] + b_ref[...]
    out_ref[...] = (jnp.dot(u1n, u2_ref[...], preferred_element_type=jnp.float32)
                    + c2_ref[...] + xv)


_update_call = pl.pallas_call(
    _update_body,
    out_shape=jax.ShapeDtypeStruct((N, H), jnp.float32),
)


def kernel(x, edge_index, edge_attr, W1, b1, W2, b2, Ws, bs,
           U1, c1, gamma, beta, U2, c2):
    src = edge_index[0]
    dst = edge_index[1]
    w1a, w1b, w1c = W1[:H], W1[H:2 * H], W1[2 * H:]

    sc_gather, sc_scatter = _sc_kernels()
    p, q = _pq_call(x, w1a, w1b, b1.reshape(1, H))
    ps, qd = sc_gather(p, q, src, dst)
    dnew, mw = _edge_call(ps, qd, edge_attr, w1c, W2, b2.reshape(1, H),
                          Ws.reshape(1, H), bs.reshape(1, 1))
    zeros = jnp.zeros((8, H), jnp.float32)
    partial = sc_scatter(mw, dst, zeros)
    feat = _update_call(partial, x, U1, c1.reshape(1, H), gamma.reshape(1, H),
                        beta.reshape(1, H), U2, c2.reshape(1, H))
    return feat, dnew


# speedup vs baseline: 3.4826x; 1.3902x over previous
import functools

import jax
import jax.numpy as jnp
from jax import lax
from jax.experimental import pallas as pl
from jax.experimental.pallas import tpu as pltpu
from jax.experimental.pallas import tpu_sc as plsc

N = 10000
E = 320000
H = 128

NC = 2
NS = 16
NW = NC * NS
EPW = E // NW
CHUNK = 80
NCHUNK = EPW // CHUNK
NSLOT = 4
RPTA = 632
RPTT = N - (NS - 1) * RPTA

@functools.cache
def _sc_kernels():
    mesh = plsc.VectorSubcoreMesh(core_axis_name="c", subcore_axis_name="s",
                                  num_cores=NC, num_subcores=NS)

    @functools.partial(
        pl.kernel,
        out_type=[
            jax.ShapeDtypeStruct((E, H), jnp.float32),
            jax.ShapeDtypeStruct((E, H), jnp.float32),
        ],
        mesh=mesh,
        scratch_types=[
            pltpu.VMEM((EPW,), jnp.int32),
            pltpu.VMEM((EPW,), jnp.int32),
            [pltpu.VMEM((CHUNK, H), jnp.float32)] * NSLOT,
            [pltpu.VMEM((CHUNK, H), jnp.float32)] * NSLOT,
            [pltpu.SemaphoreType.DMA] * NSLOT,
            [pltpu.SemaphoreType.DMA] * NSLOT,
        ],
    )
    def _sc_gather(p_hbm, q_hbm, src_hbm, dst_hbm, ps_hbm, qd_hbm,
                   idx_s, idx_d, bufs_p, bufs_q, sems_g, sems_w):
        c = lax.axis_index("c")
        s = lax.axis_index("s")
        base = (s * NC + c) * EPW

        pltpu.sync_copy(src_hbm.at[pl.ds(base, EPW)], idx_s)
        pltpu.sync_copy(dst_hbm.at[pl.ds(base, EPW)], idx_d)

        def start_g(i, b):
            ioff = i * CHUNK
            pltpu.async_copy(p_hbm.at[idx_s.at[pl.ds(ioff, CHUNK)]],
                             bufs_p[b], sems_g[b])
            pltpu.async_copy(q_hbm.at[idx_d.at[pl.ds(ioff, CHUNK)]],
                             bufs_q[b], sems_g[b])

        def wait_g(b):
            pltpu.make_async_copy(p_hbm.at[pl.ds(0, CHUNK)], bufs_p[b],
                                  sems_g[b]).wait()
            pltpu.make_async_copy(q_hbm.at[pl.ds(0, CHUNK)], bufs_q[b],
                                  sems_g[b]).wait()

        def start_w(i, b):
            off = base + i * CHUNK
            pltpu.async_copy(bufs_p[b], ps_hbm.at[pl.ds(off, CHUNK)],
                             sems_w[b])
            pltpu.async_copy(bufs_q[b], qd_hbm.at[pl.ds(off, CHUNK)],
                             sems_w[b])

        def wait_w(b):
            pltpu.make_async_copy(bufs_p[b], ps_hbm.at[pl.ds(0, CHUNK)],
                                  sems_w[b]).wait()
            pltpu.make_async_copy(bufs_q[b], qd_hbm.at[pl.ds(0, CHUNK)],
                                  sems_w[b]).wait()

        for b in range(NSLOT):
            start_g(b, b)

        def body(j, carry):
            i0 = j * NSLOT
            for b in range(NSLOT):
                wait_g(b)
                start_w(i0 + b, b)
            for b in range(NSLOT):
                wait_w(b)
                start_g(jnp.minimum(i0 + b + NSLOT, NCHUNK - 1), b)
            return carry

        lax.fori_loop(0, (NCHUNK - 1) // NSLOT, body, 0)
        for b in range(NSLOT):
            wait_g(b)
        start_w(NCHUNK - 1, 0)
        wait_w(0)

    @functools.partial(
        pl.kernel,
        out_type=jax.ShapeDtypeStruct((NC, N, H), jnp.float32),
        mesh=mesh,
        scratch_types=[
            [pltpu.VMEM((CHUNK,), jnp.int32)] * NSLOT,
            [pltpu.VMEM((CHUNK, H), jnp.float32)] * NSLOT,
            pltpu.VMEM((8, H), jnp.float32),
            pltpu.VMEM_SHARED((N, H), jnp.float32),
            [pltpu.SemaphoreType.DMA] * NSLOT,
            [pltpu.SemaphoreType.DMA] * NSLOT,
        ],
    )
    def _sc_scatter(mw_hbm, dst_hbm, z_hbm, out_hbm, idxs, bufs, zbuf, acc,
                    sems_l, sems_a):
        c = lax.axis_index("c")
        s = lax.axis_index("s")
        base = (s * NC + c) * EPW
        last = s == NS - 1

        pltpu.sync_copy(z_hbm, zbuf)
        nz = jnp.where(last, RPTT // 8, RPTA // 8)

        def zstep(k, carry):
            pltpu.sync_copy(zbuf, acc.at[pl.ds(s * RPTA + k * 8, 8)])
            return carry

        lax.fori_loop(0, nz, zstep, 0)
        plsc.subcore_barrier()

        def start_l(i, b):
            off = base + i * CHUNK
            pltpu.async_copy(dst_hbm.at[pl.ds(off, CHUNK)], idxs[b],
                             sems_l[b])
            pltpu.async_copy(mw_hbm.at[pl.ds(off, CHUNK)], bufs[b],
                             sems_l[b])

        def wait_l(b):
            pltpu.make_async_copy(dst_hbm.at[pl.ds(0, CHUNK)], idxs[b],
                                  sems_l[b]).wait()
            pltpu.make_async_copy(mw_hbm.at[pl.ds(0, CHUNK)], bufs[b],
                                  sems_l[b]).wait()

        def start_a(b):
            pltpu.async_copy(bufs[b], acc.at[idxs[b]], sems_a[b], add=True)

        def wait_a(b):
            pltpu.make_async_copy(mw_hbm.at[pl.ds(0, CHUNK)], bufs[b],
                                  sems_a[b]).wait()

        for b in range(NSLOT):
            start_l(b, b)

        def body(j, carry):
            i0 = j * NSLOT
            for b in range(NSLOT):
                wait_l(b)
                start_a(b)
            for b in range(NSLOT):
                wait_a(b)
                start_l(jnp.minimum(i0 + b + NSLOT, NCHUNK - 1), b)
            return carry

        lax.fori_loop(0, (NCHUNK - 1) // NSLOT, body, 0)
        for b in range(NSLOT):
            wait_l(b)
        start_a(0)
        wait_a(0)
        plsc.subcore_barrier()

        @pl.when(jnp.logical_not(last))
        def _():
            pltpu.sync_copy(acc.at[pl.ds(s * RPTA, RPTA)],
                            out_hbm.at[c, pl.ds(s * RPTA, RPTA)])

        @pl.when(last)
        def _():
            pltpu.sync_copy(acc.at[pl.ds(s * RPTA, RPTT)],
                            out_hbm.at[c, pl.ds(s * RPTA, RPTT)])

    return _sc_gather, _sc_scatter


def _pq_body(x_ref, w1a_ref, w1b_ref, b1_ref, p_ref, q_ref):
    xv = x_ref[...]
    p_ref[...] = (jnp.dot(xv, w1a_ref[...], preferred_element_type=jnp.float32)
                  + b1_ref[...])
    q_ref[...] = jnp.dot(xv, w1b_ref[...], preferred_element_type=jnp.float32)


_pq_call = pl.pallas_call(
    _pq_body,
    out_shape=[
        jax.ShapeDtypeStruct((N, H), jnp.float32),
        jax.ShapeDtypeStruct((N, H), jnp.float32),
    ],
)


BE = 3200


def _edge_body(ps_ref, qd_ref, ea_ref, w1c_ref, w2_ref, b2_ref, wsr_ref,
               bs_ref, dnew_ref, mw_ref):
    ea = ea_ref[...]
    h1 = jnp.maximum(
        ps_ref[...] + qd_ref[...]
        + jnp.dot(ea, w1c_ref[...], preferred_element_type=jnp.float32), 0.0)
    m = jnp.dot(h1, w2_ref[...], preferred_element_type=jnp.float32) + b2_ref[...]
    msg = m * jax.nn.sigmoid(m)
    dnew_ref[...] = ea + msg
    ew = jax.nn.sigmoid(jnp.sum(msg * wsr_ref[...], axis=1, keepdims=True)
                        + bs_ref[...])
    mw_ref[...] = msg * ew


_edge_call = pl.pallas_call(
    _edge_body,
    grid=(E // BE,),
    in_specs=[
        pl.BlockSpec((BE, H), lambda i: (i, 0)),
        pl.BlockSpec((BE, H), lambda i: (i, 0)),
        pl.BlockSpec((BE, H), lambda i: (i, 0)),
        pl.BlockSpec((H, H), lambda i: (0, 0)),
        pl.BlockSpec((H, H), lambda i: (0, 0)),
        pl.BlockSpec((1, H), lambda i: (0, 0)),
        pl.BlockSpec((1, H), lambda i: (0, 0)),
        pl.BlockSpec((1, 1), lambda i: (0, 0)),
    ],
    out_specs=[
        pl.BlockSpec((BE, H), lambda i: (i, 0)),
        pl.BlockSpec((BE, H), lambda i: (i, 0)),
    ],
    out_shape=[
        jax.ShapeDtypeStruct((E, H), jnp.float32),
        jax.ShapeDtypeStruct((E, H), jnp.float32),
    ],
)


def _update_body(part_ref, x_ref, u1_ref, c1_ref, g_ref, b_ref, u2_ref,
                 c2_ref, out_ref):
    xv = x_ref[...]
    u_in = part_ref[0] + part_ref[1] + xv
    u1 = jnp.maximum(
        jnp.dot(u_in, u1_ref[...], preferred_element_type=jnp.float32)
        + c1_ref[...], 0.0)
    mean = jnp.mean(u1, axis=0, keepdims=True)
    var = jnp.mean((u1 - mean) ** 2, axis=0, keepdims=True)
    u1n = (u1 - mean) / jnp.sqrt(var + 1e-5) * g_ref[...] + b_ref[...]
    out_ref[...] = (jnp.dot(u1n, u2_ref[...], preferred_element_type=jnp.float32)
                    + c2_ref[...] + xv)


_update_call = pl.pallas_call(
    _update_body,
    out_shape=jax.ShapeDtypeStruct((N, H), jnp.float32),
)


def kernel(x, edge_index, edge_attr, W1, b1, W2, b2, Ws, bs,
           U1, c1, gamma, beta, U2, c2):
    src = edge_index[0]
    dst = edge_index[1]
    w1a, w1b, w1c = W1[:H], W1[H:2 * H], W1[2 * H:]

    sc_gather, sc_scatter = _sc_kernels()
    p, q = _pq_call(x, w1a, w1b, b1.reshape(1, H))
    ps, qd = sc_gather(p, q, src, dst)
    dnew, mw = _edge_call(ps, qd, edge_attr, w1c, W2, b2.reshape(1, H),
                          Ws.reshape(1, H), bs.reshape(1, 1))
    zeros = jnp.zeros((8, H), jnp.float32)
    partial = sc_scatter(mw, dst, zeros)
    feat = _update_call(partial, x, U1, c1.reshape(1, H), gamma.reshape(1, H),
                        beta.reshape(1, H), U2, c2.reshape(1, H))
    return feat, dnew


# speedup vs baseline: 3.7784x; 1.0850x over previous
import functools

import jax
import jax.numpy as jnp
from jax import lax
from jax.experimental import pallas as pl
from jax.experimental.pallas import tpu as pltpu
from jax.experimental.pallas import tpu_sc as plsc

N = 10000
E = 320000
H = 128

NC = 2
NS = 16
NW = NC * NS
EPW = E // NW
CHUNK = 80
NCHUNK = EPW // CHUNK
NSLOT = 4
RPTA = 632
RPTT = N - (NS - 1) * RPTA

@functools.cache
def _sc_kernels():
    mesh = plsc.VectorSubcoreMesh(core_axis_name="c", subcore_axis_name="s",
                                  num_cores=NC, num_subcores=NS)

    @functools.partial(
        pl.kernel,
        out_type=jax.ShapeDtypeStruct((E, H), jnp.float32),
        mesh=mesh,
        scratch_types=[
            pltpu.VMEM((EPW,), jnp.int32),
            pltpu.VMEM((EPW,), jnp.int32),
            [pltpu.VMEM((CHUNK, H), jnp.float32)] * NSLOT,
            [pltpu.SemaphoreType.DMA] * NSLOT,
            [pltpu.SemaphoreType.DMA] * NSLOT,
            [pltpu.SemaphoreType.DMA] * NSLOT,
        ],
    )
    def _sc_gather(p_hbm, q_hbm, src_hbm, dst_hbm, g_hbm,
                   idx_s, idx_d, bufs, sems_p, sems_a, sems_w):
        c = lax.axis_index("c")
        s = lax.axis_index("s")
        base = (s * NC + c) * EPW

        pltpu.sync_copy(src_hbm.at[pl.ds(base, EPW)], idx_s)
        pltpu.sync_copy(dst_hbm.at[pl.ds(base, EPW)], idx_d)

        def start_p(i, b):
            pltpu.async_copy(p_hbm.at[idx_s.at[pl.ds(i * CHUNK, CHUNK)]],
                             bufs[b], sems_p[b])

        def wait_p(b):
            pltpu.make_async_copy(p_hbm.at[pl.ds(0, CHUNK)], bufs[b],
                                  sems_p[b]).wait()

        def start_a(i, b):
            pltpu.async_copy(q_hbm.at[idx_d.at[pl.ds(i * CHUNK, CHUNK)]],
                             bufs[b], sems_a[b], add=True)

        def wait_a(b):
            pltpu.make_async_copy(q_hbm.at[pl.ds(0, CHUNK)], bufs[b],
                                  sems_a[b]).wait()

        def start_w(i, b):
            off = base + i * CHUNK
            pltpu.async_copy(bufs[b], g_hbm.at[pl.ds(off, CHUNK)],
                             sems_w[b])

        def wait_w(b):
            pltpu.make_async_copy(bufs[b], g_hbm.at[pl.ds(0, CHUNK)],
                                  sems_w[b]).wait()

        for b in range(NSLOT):
            start_p(b, b)

        def body(j, carry):
            i0 = j * NSLOT
            for b in range(NSLOT):
                wait_p(b)
                start_a(i0 + b, b)
            for b in range(NSLOT):
                wait_a(b)
                start_w(i0 + b, b)
            for b in range(NSLOT):
                wait_w(b)
                start_p(jnp.minimum(i0 + b + NSLOT, NCHUNK - 1), b)
            return carry

        lax.fori_loop(0, (NCHUNK - 1) // NSLOT, body, 0)
        wait_p(0)
        start_a(NCHUNK - 1, 0)
        for b in range(1, NSLOT):
            wait_p(b)
        wait_a(0)
        start_w(NCHUNK - 1, 0)
        wait_w(0)

    @functools.partial(
        pl.kernel,
        out_type=jax.ShapeDtypeStruct((NC, N, H), jnp.float32),
        mesh=mesh,
        scratch_types=[
            [pltpu.VMEM((CHUNK,), jnp.int32)] * NSLOT,
            [pltpu.VMEM((CHUNK, H), jnp.float32)] * NSLOT,
            pltpu.VMEM((8, H), jnp.float32),
            pltpu.VMEM_SHARED((N, H), jnp.float32),
            [pltpu.SemaphoreType.DMA] * NSLOT,
            [pltpu.SemaphoreType.DMA] * NSLOT,
        ],
    )
    def _sc_scatter(mw_hbm, dst_hbm, z_hbm, out_hbm, idxs, bufs, zbuf, acc,
                    sems_l, sems_a):
        c = lax.axis_index("c")
        s = lax.axis_index("s")
        base = (s * NC + c) * EPW
        last = s == NS - 1

        pltpu.sync_copy(z_hbm, zbuf)
        nz = jnp.where(last, RPTT // 8, RPTA // 8)

        def zstep(k, carry):
            pltpu.sync_copy(zbuf, acc.at[pl.ds(s * RPTA + k * 8, 8)])
            return carry

        lax.fori_loop(0, nz, zstep, 0)
        plsc.subcore_barrier()

        def start_l(i, b):
            off = base + i * CHUNK
            pltpu.async_copy(dst_hbm.at[pl.ds(off, CHUNK)], idxs[b],
                             sems_l[b])
            pltpu.async_copy(mw_hbm.at[pl.ds(off, CHUNK)], bufs[b],
                             sems_l[b])

        def wait_l(b):
            pltpu.make_async_copy(dst_hbm.at[pl.ds(0, CHUNK)], idxs[b],
                                  sems_l[b]).wait()
            pltpu.make_async_copy(mw_hbm.at[pl.ds(0, CHUNK)], bufs[b],
                                  sems_l[b]).wait()

        def start_a(b):
            pltpu.async_copy(bufs[b], acc.at[idxs[b]], sems_a[b], add=True)

        def wait_a(b):
            pltpu.make_async_copy(mw_hbm.at[pl.ds(0, CHUNK)], bufs[b],
                                  sems_a[b]).wait()

        for b in range(NSLOT):
            start_l(b, b)

        def body(j, carry):
            i0 = j * NSLOT
            for b in range(NSLOT):
                wait_l(b)
                start_a(b)
            for b in range(NSLOT):
                wait_a(b)
                start_l(jnp.minimum(i0 + b + NSLOT, NCHUNK - 1), b)
            return carry

        lax.fori_loop(0, (NCHUNK - 1) // NSLOT, body, 0)
        for b in range(NSLOT):
            wait_l(b)
        start_a(0)
        wait_a(0)
        plsc.subcore_barrier()

        @pl.when(jnp.logical_not(last))
        def _():
            pltpu.sync_copy(acc.at[pl.ds(s * RPTA, RPTA)],
                            out_hbm.at[c, pl.ds(s * RPTA, RPTA)])

        @pl.when(last)
        def _():
            pltpu.sync_copy(acc.at[pl.ds(s * RPTA, RPTT)],
                            out_hbm.at[c, pl.ds(s * RPTA, RPTT)])

    return _sc_gather, _sc_scatter


def _pq_body(x_ref, w1a_ref, w1b_ref, b1_ref, p_ref, q_ref):
    xv = x_ref[...]
    p_ref[...] = (jnp.dot(xv, w1a_ref[...], preferred_element_type=jnp.float32)
                  + b1_ref[...])
    q_ref[...] = jnp.dot(xv, w1b_ref[...], preferred_element_type=jnp.float32)


_pq_call = pl.pallas_call(
    _pq_body,
    out_shape=[
        jax.ShapeDtypeStruct((N, H), jnp.float32),
        jax.ShapeDtypeStruct((N, H), jnp.float32),
    ],
)


BE = 3200


def _edge_body(g_ref, ea_ref, w1c_ref, w2_ref, b2_ref, wsr_ref,
               bs_ref, dnew_ref, mw_ref):
    ea = ea_ref[...]
    h1 = jnp.maximum(
        g_ref[...]
        + jnp.dot(ea, w1c_ref[...], preferred_element_type=jnp.float32), 0.0)
    m = jnp.dot(h1, w2_ref[...], preferred_element_type=jnp.float32) + b2_ref[...]
    msg = m / (1.0 + jnp.exp(-m))
    dnew_ref[...] = ea + msg
    g = jnp.sum(msg * wsr_ref[...], axis=1, keepdims=True) + bs_ref[...]
    ew = 1.0 / (1.0 + jnp.exp(-g))
    mw_ref[...] = msg * ew


_edge_call = pl.pallas_call(
    _edge_body,
    grid=(E // BE,),
    in_specs=[
        pl.BlockSpec((BE, H), lambda i: (i, 0)),
        pl.BlockSpec((BE, H), lambda i: (i, 0)),
        pl.BlockSpec((H, H), lambda i: (0, 0)),
        pl.BlockSpec((H, H), lambda i: (0, 0)),
        pl.BlockSpec((1, H), lambda i: (0, 0)),
        pl.BlockSpec((1, H), lambda i: (0, 0)),
        pl.BlockSpec((1, 1), lambda i: (0, 0)),
    ],
    out_specs=[
        pl.BlockSpec((BE, H), lambda i: (i, 0)),
        pl.BlockSpec((BE, H), lambda i: (i, 0)),
    ],
    out_shape=[
        jax.ShapeDtypeStruct((E, H), jnp.float32),
        jax.ShapeDtypeStruct((E, H), jnp.float32),
    ],
)


def _update_body(part_ref, x_ref, u1_ref, c1_ref, g_ref, b_ref, u2_ref,
                 c2_ref, out_ref):
    xv = x_ref[...]
    u_in = part_ref[0] + part_ref[1] + xv
    u1 = jnp.maximum(
        jnp.dot(u_in, u1_ref[...], preferred_element_type=jnp.float32)
        + c1_ref[...], 0.0)
    mean = jnp.mean(u1, axis=0, keepdims=True)
    var = jnp.mean((u1 - mean) ** 2, axis=0, keepdims=True)
    u1n = (u1 - mean) / jnp.sqrt(var + 1e-5) * g_ref[...] + b_ref[...]
    out_ref[...] = (jnp.dot(u1n, u2_ref[...], preferred_element_type=jnp.float32)
                    + c2_ref[...] + xv)


_update_call = pl.pallas_call(
    _update_body,
    out_shape=jax.ShapeDtypeStruct((N, H), jnp.float32),
)


def kernel(x, edge_index, edge_attr, W1, b1, W2, b2, Ws, bs,
           U1, c1, gamma, beta, U2, c2):
    src = edge_index[0]
    dst = edge_index[1]
    w1a, w1b, w1c = W1[:H], W1[H:2 * H], W1[2 * H:]

    sc_gather, sc_scatter = _sc_kernels()
    p, q = _pq_call(x, w1a, w1b, b1.reshape(1, H))
    g = sc_gather(p, q, src, dst)
    dnew, mw = _edge_call(g, edge_attr, w1c, W2, b2.reshape(1, H),
                          Ws.reshape(1, H), bs.reshape(1, 1))
    zeros = jnp.zeros((8, H), jnp.float32)
    partial = sc_scatter(mw, dst, zeros)
    feat = _update_call(partial, x, U1, c1.reshape(1, H), gamma.reshape(1, H),
                        beta.reshape(1, H), U2, c2.reshape(1, H))
    return feat, dnew


# speedup vs baseline: 4.3152x; 1.1421x over previous
import functools

import jax
import jax.numpy as jnp
from jax import lax
from jax.experimental import pallas as pl
from jax.experimental.pallas import tpu as pltpu
from jax.experimental.pallas import tpu_sc as plsc

N = 10000
E = 320000
H = 128

NSLAB = 2
ES = E // NSLAB
NC = 2
NS = 16
NW = NC * NS
EPW = ES // NW
CHUNK = 40
NCHUNK = EPW // CHUNK
NSLOT = 4
RPTA = 632
RPTT = N - (NS - 1) * RPTA


@functools.cache
def _sc_kernels():
    mesh = plsc.VectorSubcoreMesh(core_axis_name="c", subcore_axis_name="s",
                                  num_cores=NC, num_subcores=NS)

    def _make_gather(slab):
        @functools.partial(
            pl.kernel,
            out_type=jax.ShapeDtypeStruct((ES, H), jnp.float32),
            mesh=mesh,
            scratch_types=[
                pltpu.VMEM((EPW,), jnp.int32),
                pltpu.VMEM((EPW,), jnp.int32),
                [pltpu.VMEM((CHUNK, H), jnp.float32)] * NSLOT,
                [pltpu.SemaphoreType.DMA] * NSLOT,
                [pltpu.SemaphoreType.DMA] * NSLOT,
                [pltpu.SemaphoreType.DMA] * NSLOT,
            ],
        )
        def _sc_gather(p_hbm, q_hbm, src_hbm, dst_hbm, g_hbm,
                       idx_s, idx_d, bufs, sems_p, sems_a, sems_w):
            c = lax.axis_index("c")
            s = lax.axis_index("s")
            lbase = (s * NC + c) * EPW
            ebase = slab * ES + lbase

            pltpu.sync_copy(src_hbm.at[pl.ds(ebase, EPW)], idx_s)
            pltpu.sync_copy(dst_hbm.at[pl.ds(ebase, EPW)], idx_d)

            def start_p(i, b):
                pltpu.async_copy(p_hbm.at[idx_s.at[pl.ds(i * CHUNK, CHUNK)]],
                                 bufs[b], sems_p[b])

            def wait_p(b):
                pltpu.make_async_copy(p_hbm.at[pl.ds(0, CHUNK)], bufs[b],
                                      sems_p[b]).wait()

            def start_a(i, b):
                pltpu.async_copy(q_hbm.at[idx_d.at[pl.ds(i * CHUNK, CHUNK)]],
                                 bufs[b], sems_a[b], add=True)

            def wait_a(b):
                pltpu.make_async_copy(q_hbm.at[pl.ds(0, CHUNK)], bufs[b],
                                      sems_a[b]).wait()

            def start_w(i, b):
                pltpu.async_copy(bufs[b], g_hbm.at[pl.ds(lbase + i * CHUNK,
                                                         CHUNK)], sems_w[b])

            def wait_w(b):
                pltpu.make_async_copy(bufs[b], g_hbm.at[pl.ds(0, CHUNK)],
                                      sems_w[b]).wait()

            for b in range(NSLOT):
                start_p(b, b)

            def body(j, carry):
                i0 = j * NSLOT
                for b in range(NSLOT):
                    wait_p(b)
                    start_a(i0 + b, b)
                for b in range(NSLOT):
                    wait_a(b)
                    start_w(i0 + b, b)
                for b in range(NSLOT):
                    wait_w(b)
                    start_p(jnp.minimum(i0 + b + NSLOT, NCHUNK - 1), b)
                return carry

            lax.fori_loop(0, (NCHUNK - 1) // NSLOT, body, 0)
            wait_p(0)
            start_a(NCHUNK - 1, 0)
            for b in range(1, NSLOT):
                wait_p(b)
            wait_a(0)
            start_w(NCHUNK - 1, 0)
            wait_w(0)

        return _sc_gather

    def _make_scatter(slab):
        @functools.partial(
            pl.kernel,
            out_type=jax.ShapeDtypeStruct((NC, N, H), jnp.float32),
            mesh=mesh,
            scratch_types=[
                [pltpu.VMEM((CHUNK,), jnp.int32)] * NSLOT,
                [pltpu.VMEM((CHUNK, H), jnp.float32)] * NSLOT,
                pltpu.VMEM((8, H), jnp.float32),
                pltpu.VMEM_SHARED((N, H), jnp.float32),
                [pltpu.SemaphoreType.DMA] * NSLOT,
                [pltpu.SemaphoreType.DMA] * NSLOT,
            ],
        )
        def _sc_scatter(mw_hbm, dst_hbm, z_hbm, out_hbm, idxs, bufs, zbuf,
                        acc, sems_l, sems_a):
            c = lax.axis_index("c")
            s = lax.axis_index("s")
            lbase = (s * NC + c) * EPW
            ebase = slab * ES + lbase
            last = s == NS - 1

            pltpu.sync_copy(z_hbm, zbuf)
            nz = jnp.where(last, RPTT // 8, RPTA // 8)

            def zstep(k, carry):
                pltpu.sync_copy(zbuf, acc.at[pl.ds(s * RPTA + k * 8, 8)])
                return carry

            lax.fori_loop(0, nz, zstep, 0)
            plsc.subcore_barrier()

            def start_l(i, b):
                pltpu.async_copy(dst_hbm.at[pl.ds(ebase + i * CHUNK, CHUNK)],
                                 idxs[b], sems_l[b])
                pltpu.async_copy(mw_hbm.at[pl.ds(lbase + i * CHUNK, CHUNK)],
                                 bufs[b], sems_l[b])

            def wait_l(b):
                pltpu.make_async_copy(dst_hbm.at[pl.ds(0, CHUNK)], idxs[b],
                                      sems_l[b]).wait()
                pltpu.make_async_copy(mw_hbm.at[pl.ds(0, CHUNK)], bufs[b],
                                      sems_l[b]).wait()

            def start_a(b):
                pltpu.async_copy(bufs[b], acc.at[idxs[b]], sems_a[b],
                                 add=True)

            def wait_a(b):
                pltpu.make_async_copy(mw_hbm.at[pl.ds(0, CHUNK)], bufs[b],
                                      sems_a[b]).wait()

            for b in range(NSLOT):
                start_l(b, b)

            def body(j, carry):
                i0 = j * NSLOT
                for b in range(NSLOT):
                    wait_l(b)
                    start_a(b)
                for b in range(NSLOT):
                    wait_a(b)
                    start_l(jnp.minimum(i0 + b + NSLOT, NCHUNK - 1), b)
                return carry

            lax.fori_loop(0, (NCHUNK - 1) // NSLOT, body, 0)
            for b in range(NSLOT):
                wait_l(b)
            start_a(0)
            wait_a(0)
            plsc.subcore_barrier()

            @pl.when(jnp.logical_not(last))
            def _():
                pltpu.sync_copy(acc.at[pl.ds(s * RPTA, RPTA)],
                                out_hbm.at[c, pl.ds(s * RPTA, RPTA)])

            @pl.when(last)
            def _():
                pltpu.sync_copy(acc.at[pl.ds(s * RPTA, RPTT)],
                                out_hbm.at[c, pl.ds(s * RPTA, RPTT)])

        return _sc_scatter

    return ([_make_gather(k) for k in range(NSLAB)],
            [_make_scatter(k) for k in range(NSLAB)])


def _pq_body(x_ref, w1a_ref, w1b_ref, b1_ref, p_ref, q_ref):
    xv = x_ref[...]
    p_ref[...] = (jnp.dot(xv, w1a_ref[...], preferred_element_type=jnp.float32)
                  + b1_ref[...])
    q_ref[...] = jnp.dot(xv, w1b_ref[...], preferred_element_type=jnp.float32)


_pq_call = pl.pallas_call(
    _pq_body,
    out_shape=[
        jax.ShapeDtypeStruct((N, H), jnp.float32),
        jax.ShapeDtypeStruct((N, H), jnp.float32),
    ],
)


BE = 3200
GS = ES // BE


def _edge_math(g_ref, ea_ref, w1c_ref, w2_ref, b2_ref, wsr_ref, bs_ref,
               dnew_ref, mw_ref):
    ea = ea_ref[...]
    h1 = jnp.maximum(
        g_ref[...]
        + jnp.dot(ea, w1c_ref[...], preferred_element_type=jnp.float32), 0.0)
    m = jnp.dot(h1, w2_ref[...], preferred_element_type=jnp.float32) + b2_ref[...]
    msg = m / (1.0 + jnp.exp(-m))
    dnew_ref[...] = ea + msg
    gl = jnp.sum(msg * wsr_ref[...], axis=1, keepdims=True) + bs_ref[...]
    ew = 1.0 / (1.0 + jnp.exp(-gl))
    mw_ref[...] = msg * ew


def _make_edge_call(slab):
    def body0(g_ref, ea_ref, w1c_ref, w2_ref, b2_ref, wsr_ref, bs_ref,
              dnew_ref, mw_ref):
        _edge_math(g_ref, ea_ref, w1c_ref, w2_ref, b2_ref, wsr_ref, bs_ref,
                   dnew_ref, mw_ref)

    def body1(g_ref, ea_ref, w1c_ref, w2_ref, b2_ref, wsr_ref, bs_ref,
              prev_ref, dnew_ref, mw_ref):
        _edge_math(g_ref, ea_ref, w1c_ref, w2_ref, b2_ref, wsr_ref, bs_ref,
                   dnew_ref, mw_ref)

    in_specs = [
        pl.BlockSpec((BE, H), lambda i: (i, 0)),
        pl.BlockSpec((BE, H), lambda i, k=slab: (i + k * GS, 0)),
        pl.BlockSpec((H, H), lambda i: (0, 0)),
        pl.BlockSpec((H, H), lambda i: (0, 0)),
        pl.BlockSpec((1, H), lambda i: (0, 0)),
        pl.BlockSpec((1, H), lambda i: (0, 0)),
        pl.BlockSpec((1, 1), lambda i: (0, 0)),
    ]
    kwargs = {}
    if slab == 1:
        in_specs = in_specs + [pl.BlockSpec((8, H), lambda i: (0, 0))]
        kwargs["input_output_aliases"] = {7: 0}
    return pl.pallas_call(
        body1 if slab == 1 else body0,
        grid=(GS,),
        in_specs=in_specs,
        out_specs=[
            pl.BlockSpec((BE, H), lambda i, k=slab: (i + k * GS, 0)),
            pl.BlockSpec((BE, H), lambda i: (i, 0)),
        ],
        out_shape=[
            jax.ShapeDtypeStruct((E, H), jnp.float32),
            jax.ShapeDtypeStruct((ES, H), jnp.float32),
        ],
        **kwargs,
    )


_edge_calls = [_make_edge_call(k) for k in range(NSLAB)]


def _update_body(p0_ref, p1_ref, x_ref, u1_ref, c1_ref, g_ref, b_ref,
                 u2_ref, c2_ref, out_ref):
    xv = x_ref[...]
    u_in = p0_ref[0] + p0_ref[1] + p1_ref[0] + p1_ref[1] + xv
    u1 = jnp.maximum(
        jnp.dot(u_in, u1_ref[...], preferred_element_type=jnp.float32)
        + c1_ref[...], 0.0)
    mean = jnp.mean(u1, axis=0, keepdims=True)
    var = jnp.mean((u1 - mean) ** 2, axis=0, keepdims=True)
    u1n = (u1 - mean) / jnp.sqrt(var + 1e-5) * g_ref[...] + b_ref[...]
    out_ref[...] = (jnp.dot(u1n, u2_ref[...], preferred_element_type=jnp.float32)
                    + c2_ref[...] + xv)


_update_call = pl.pallas_call(
    _update_body,
    out_shape=jax.ShapeDtypeStruct((N, H), jnp.float32),
)


def kernel(x, edge_index, edge_attr, W1, b1, W2, b2, Ws, bs,
           U1, c1, gamma, beta, U2, c2):
    src = edge_index[0]
    dst = edge_index[1]
    w1a, w1b, w1c = W1[:H], W1[H:2 * H], W1[2 * H:]

    gathers, scatters = _sc_kernels()
    p, q = _pq_call(x, w1a, w1b, b1.reshape(1, H))
    b2r, wsr, bsr = b2.reshape(1, H), Ws.reshape(1, H), bs.reshape(1, 1)
    zeros = jnp.zeros((8, H), jnp.float32)

    g0 = gathers[0](p, q, src, dst)
    g1 = gathers[1](p, q, src, dst)
    dnew0, mw0 = _edge_calls[0](g0, edge_attr, w1c, W2, b2r, wsr, bsr)
    part0 = scatters[0](mw0, dst, zeros)
    dnew, mw1 = _edge_calls[1](g1, edge_attr, w1c, W2, b2r, wsr, bsr, dnew0)
    part1 = scatters[1](mw1, dst, zeros)
    feat = _update_call(part0, part1, x, U1, c1.reshape(1, H),
                        gamma.reshape(1, H), beta.reshape(1, H), U2,
                        c2.reshape(1, H))
    return feat, dnew
